# baseline (speedup 1.0000x reference)
import functools

import jax
import jax.numpy as jnp
from jax import lax
from jax.experimental import pallas as pl
from jax.experimental.pallas import tpu as pltpu
from jax.experimental.pallas import tpu_sc as plsc

B, N, S = 4, 8192, 2048
D1, D2 = 16, 32
H1, H2 = 64, 64

NB1 = 512
NB2 = 2048
NH = N // 2
NSEL = 3 * B * NH
DPAD = 128
SC_CHUNK = 128


def _k1sel_body(dist_ref, wout_ref, iout_ref):
    d = dist_ref[0]
    iota = lax.broadcasted_iota(jnp.int32, (NB1, S), 1).astype(jnp.float32)
    base = (pl.program_id(0) * S).astype(jnp.int32)
    recips = []
    idxs = []
    for k in range(3):
        vmin = jnp.min(d, axis=1, keepdims=True)
        t = jnp.where(d == vmin, iota, float(S))
        imin = jnp.min(t, axis=1, keepdims=True)
        recips.append(1.0 / (vmin + 1e-8))
        idxs.append(imin + base.astype(jnp.float32))
        if k < 2:
            d = jnp.where(t == imin, jnp.inf, d)
    norm = (recips[0] + recips[1]) + recips[2]
    z = jnp.zeros((NB1, 5), jnp.float32)
    wout_ref[0, 0] = jnp.concatenate(
        [recips[0] / norm, recips[1] / norm, recips[2] / norm, z], axis=1).T
    iout_ref[0, 0] = jnp.concatenate(idxs + [z], axis=1).T


_SC_INFO = plsc.get_sparse_core_info()
_NW = _SC_INFO.num_cores * _SC_INFO.num_subcores


def _sc_gather_body(table_hbm, idx_hbm, out_hbm, idx_v, rows_v, sem):
    wid = lax.axis_index("s") * _SC_INFO.num_cores + lax.axis_index("c")
    per_w = NSEL // _NW
    n_chunks = per_w // SC_CHUNK
    for j in range(n_chunks):
        base = wid * per_w + j * SC_CHUNK
        pltpu.sync_copy(idx_hbm.at[pl.ds(base, SC_CHUNK)], idx_v)
        pltpu.async_copy(table_hbm.at[idx_v], rows_v, sem).wait()
        pltpu.sync_copy(rows_v, out_hbm.at[pl.ds(base, SC_CHUNK)])


_sc_gather = functools.partial(
    pl.kernel,
    mesh=plsc.VectorSubcoreMesh(core_axis_name="c", subcore_axis_name="s"),
    out_type=jax.ShapeDtypeStruct((NSEL, DPAD), jnp.float32),
    scratch_types=[
        pltpu.VMEM((SC_CHUNK,), jnp.int32),
        pltpu.VMEM((SC_CHUNK, DPAD), jnp.float32),
        pltpu.SemaphoreType.DMA,
    ],
)(_sc_gather_body)


def _k1b_body(f0_ref, f1_ref, f2_ref, w0_ref, w1_ref, w2_ref, p1_ref,
              w1a_ref, w1b_ref, b1_ref, y1_ref, sums_ref):
    interp = ((w0_ref[0] * f0_ref[0] + w1_ref[0] * f1_ref[0])
              + w2_ref[0] * f2_ref[0])
    y1 = (lax.dot_general(p1_ref[0], w1a_ref[...], (((1,), (1,)), ((), ())),
                          preferred_element_type=jnp.float32)
          + lax.dot_general(interp, w1b_ref[...], (((1,), (1,)), ((), ())),
                            preferred_element_type=jnp.float32)
          + b1_ref[...])
    y1_ref[0] = y1
    s1 = jnp.sum(y1, axis=0, keepdims=True)
    s2 = jnp.sum(y1 * y1, axis=0, keepdims=True)
    sums_ref[0, 0] = jnp.concatenate(
        [s1, s2, jnp.zeros((6, H1), jnp.float32)], axis=0)


def _k2_body(y1_ref, sc_ref, sh_ref, w2_ref, b2_ref, y2_ref, sums_ref):
    h = jnp.maximum(y1_ref[0] * sc_ref[...] + sh_ref[...], 0.0)
    y2 = lax.dot_general(h, w2_ref[...], (((1,), (1,)), ((), ())),
                         preferred_element_type=jnp.float32) + b2_ref[...]
    y2_ref[0] = y2
    s1 = jnp.sum(y2, axis=0, keepdims=True)
    s2 = jnp.sum(y2 * y2, axis=0, keepdims=True)
    sums_ref[0, 0] = jnp.concatenate(
        [s1, s2, jnp.zeros((6, H2), jnp.float32)], axis=0)


def _k3_body(y2_ref, sc_ref, sh_ref, out_ref):
    out_ref[0] = jnp.maximum(y2_ref[0] * sc_ref[...] + sh_ref[...], 0.0)


@jax.jit
def kernel(xyz1, xyz2, points1, points2, W1, b1, g1, be1, W2, b2, g2, be2):
    nblk1 = N // NB1
    nblk2 = N // NB2

    dist = -2.0 * jnp.einsum('bnc,bmc->bnm', xyz1, xyz2)
    dist = dist + jnp.sum(xyz1 ** 2, axis=-1)[:, :, None]
    dist = dist + jnp.sum(xyz2 ** 2, axis=-1)[:, None, :]

    W1a = W1[:, :D1]
    W1b = W1[:, D1:]

    table = jnp.pad(points2.reshape(B * S, D2), ((0, 0), (0, DPAD - D2)))
    nblk_h = NH // NB1

    sel_halves = []
    for h in range(2):
        dist_h = lax.slice_in_dim(dist, h * NH, (h + 1) * NH, axis=1)
        sel_halves.append(pl.pallas_call(
            _k1sel_body,
            grid=(B, nblk_h),
            in_specs=[pl.BlockSpec((1, NB1, S), lambda b, i: (b, i, 0))],
            out_specs=[
                pl.BlockSpec((1, 1, 8, NB1), lambda b, i: (b, i, 0, 0)),
                pl.BlockSpec((1, 1, 8, NB1), lambda b, i: (b, i, 0, 0)),
            ],
            out_shape=[
                jax.ShapeDtypeStruct((B, nblk_h, 8, NB1), jnp.float32),
                jax.ShapeDtypeStruct((B, nblk_h, 8, NB1), jnp.float32),
            ],
        )(dist_h))

    rows_halves = []
    for h in range(2):
        _, isel = sel_halves[h]
        idx_all = jnp.concatenate(
            [isel[:, :, k, :].reshape(B * NH) for k in range(3)]
        ).astype(jnp.int32)
        rows_halves.append(_sc_gather(table, idx_all))

    y1_halves, sums1_list = [], []
    for h in range(2):
        wsel, _ = sel_halves[h]
        feats = rows_halves[h][:, :D2].reshape(3, B, NH, D2)
        wq = [wsel[:, :, k, :].reshape(B, NH, 1) for k in range(3)]
        p1_h = lax.slice_in_dim(points1, h * NH, (h + 1) * NH, axis=1)
        y1_h, sums1_h = pl.pallas_call(
            _k1b_body,
            grid=(B, nblk_h),
            in_specs=[
                pl.BlockSpec((1, NB1, D2), lambda b, i: (b, i, 0)),
                pl.BlockSpec((1, NB1, D2), lambda b, i: (b, i, 0)),
                pl.BlockSpec((1, NB1, D2), lambda b, i: (b, i, 0)),
                pl.BlockSpec((1, NB1, 1), lambda b, i: (b, i, 0)),
                pl.BlockSpec((1, NB1, 1), lambda b, i: (b, i, 0)),
                pl.BlockSpec((1, NB1, 1), lambda b, i: (b, i, 0)),
                pl.BlockSpec((1, NB1, D1), lambda b, i: (b, i, 0)),
                pl.BlockSpec((H1, D1), lambda b, i: (0, 0)),
                pl.BlockSpec((H1, D2), lambda b, i: (0, 0)),
                pl.BlockSpec((1, H1), lambda b, i: (0, 0)),
            ],
            out_specs=[
                pl.BlockSpec((1, NB1, H1), lambda b, i: (b, i, 0)),
                pl.BlockSpec((1, 1, 8, H1), lambda b, i: (b, i, 0, 0)),
            ],
            out_shape=[
                jax.ShapeDtypeStruct((B, NH, H1), jnp.float32),
                jax.ShapeDtypeStruct((B, nblk_h, 8, H1), jnp.float32),
            ],
        )(feats[0], feats[1], feats[2], wq[0], wq[1], wq[2], p1_h,
          W1a, W1b, b1.reshape(1, H1))
        y1_halves.append(y1_h)
        sums1_list.append(sums1_h)

    y1 = jnp.concatenate(y1_halves, axis=1)
    cnt = float(B * N)
    t = jnp.sum(sums1_list[0], axis=(0, 1)) + jnp.sum(sums1_list[1], axis=(0, 1))
    mean1, ex2 = t[0] / cnt, t[1] / cnt
    var1 = ex2 - mean1 * mean1
    sc1 = g1 / jnp.sqrt(var1 + 1e-5)
    sh1 = be1 - mean1 * sc1

    y2, sums2 = pl.pallas_call(
        _k2_body,
        grid=(B, nblk2),
        in_specs=[
            pl.BlockSpec((1, NB2, H1), lambda b, i: (b, i, 0)),
            pl.BlockSpec((1, H1), lambda b, i: (0, 0)),
            pl.BlockSpec((1, H1), lambda b, i: (0, 0)),
            pl.BlockSpec((H2, H1), lambda b, i: (0, 0)),
            pl.BlockSpec((1, H2), lambda b, i: (0, 0)),
        ],
        out_specs=[
            pl.BlockSpec((1, NB2, H2), lambda b, i: (b, i, 0)),
            pl.BlockSpec((1, 1, 8, H2), lambda b, i: (b, i, 0, 0)),
        ],
        out_shape=[
            jax.ShapeDtypeStruct((B, N, H2), jnp.float32),
            jax.ShapeDtypeStruct((B, nblk2, 8, H2), jnp.float32),
        ],
    )(y1, sc1.reshape(1, H1), sh1.reshape(1, H1), W2, b2.reshape(1, H2))

    t = jnp.sum(sums2, axis=(0, 1))
    mean2, ex2 = t[0] / cnt, t[1] / cnt
    var2 = ex2 - mean2 * mean2
    sc2 = g2 / jnp.sqrt(var2 + 1e-5)
    sh2 = be2 - mean2 * sc2

    out = pl.pallas_call(
        _k3_body,
        grid=(B, nblk2),
        in_specs=[
            pl.BlockSpec((1, NB2, H2), lambda b, i: (b, i, 0)),
            pl.BlockSpec((1, H2), lambda b, i: (0, 0)),
            pl.BlockSpec((1, H2), lambda b, i: (0, 0)),
        ],
        out_specs=pl.BlockSpec((1, NB2, H2), lambda b, i: (b, i, 0)),
        out_shape=jax.ShapeDtypeStruct((B, N, H2), jnp.float32),
    )(y2, sc2.reshape(1, H2), sh2.reshape(1, H2))

    return out


# speedup vs baseline: 1.3645x; 1.3645x over previous
import functools

import jax
import jax.numpy as jnp
from jax import lax
from jax.experimental import pallas as pl
from jax.experimental.pallas import tpu as pltpu
from jax.experimental.pallas import tpu_sc as plsc

B, N, S = 4, 8192, 2048
D1, D2 = 16, 32
H1, H2 = 64, 64

NB1 = 512
NB2 = 2048
NSEL = 3 * B * N
DPAD = 128
SC_CHUNK = 128


def _k1sel_body(dist_ref, wout_ref, iout_ref):
    d = dist_ref[0]
    iota = lax.broadcasted_iota(jnp.int32, (NB1, S), 1).astype(jnp.float32)
    base = (pl.program_id(0) * S).astype(jnp.int32)
    recips = []
    idxs = []
    for k in range(3):
        vmin = jnp.min(d, axis=1, keepdims=True)
        t = jnp.where(d == vmin, iota, float(S))
        imin = jnp.min(t, axis=1, keepdims=True)
        recips.append(1.0 / (vmin + 1e-8))
        idxs.append(imin + base.astype(jnp.float32))
        if k < 2:
            d = jnp.where(t == imin, jnp.inf, d)
    norm = (recips[0] + recips[1]) + recips[2]
    z = jnp.zeros((NB1, 5), jnp.float32)
    wout_ref[0, 0] = jnp.concatenate(
        [recips[0] / norm, recips[1] / norm, recips[2] / norm, z], axis=1).T
    iout_ref[0, 0] = jnp.concatenate(idxs + [z], axis=1).T


_SC_INFO = plsc.get_sparse_core_info()
_NW = _SC_INFO.num_cores * _SC_INFO.num_subcores


def _sc_gather_body(table_hbm, idx_hbm, out_hbm, idx_v, rows_v, sem):
    wid = lax.axis_index("s") * _SC_INFO.num_cores + lax.axis_index("c")
    per_w = NSEL // _NW
    n_chunks = per_w // SC_CHUNK
    for j in range(n_chunks):
        base = wid * per_w + j * SC_CHUNK
        pltpu.sync_copy(idx_hbm.at[pl.ds(base, SC_CHUNK)], idx_v)
        pltpu.async_copy(table_hbm.at[idx_v], rows_v, sem).wait()
        pltpu.sync_copy(rows_v, out_hbm.at[pl.ds(base, SC_CHUNK)])


_sc_gather = functools.partial(
    pl.kernel,
    mesh=plsc.VectorSubcoreMesh(core_axis_name="c", subcore_axis_name="s"),
    out_type=jax.ShapeDtypeStruct((NSEL, DPAD), jnp.float32),
    scratch_types=[
        pltpu.VMEM((SC_CHUNK,), jnp.int32),
        pltpu.VMEM((SC_CHUNK, DPAD), jnp.float32),
        pltpu.SemaphoreType.DMA,
    ],
)(_sc_gather_body)


def _k1b_body(f0_ref, f1_ref, f2_ref, w0_ref, w1_ref, w2_ref, p1_ref,
              w1a_ref, w1b_ref, b1_ref, y1_ref, sums_ref):
    interp = ((w0_ref[0] * f0_ref[0] + w1_ref[0] * f1_ref[0])
              + w2_ref[0] * f2_ref[0])
    y1 = (lax.dot_general(p1_ref[0], w1a_ref[...], (((1,), (1,)), ((), ())),
                          preferred_element_type=jnp.float32)
          + lax.dot_general(interp, w1b_ref[...], (((1,), (1,)), ((), ())),
                            preferred_element_type=jnp.float32)
          + b1_ref[...])
    y1_ref[0] = y1
    s1 = jnp.sum(y1, axis=0, keepdims=True)
    s2 = jnp.sum(y1 * y1, axis=0, keepdims=True)
    sums_ref[0, 0] = jnp.concatenate(
        [s1, s2, jnp.zeros((6, H1), jnp.float32)], axis=0)


def _k2_body(y1_ref, sc_ref, sh_ref, w2_ref, b2_ref, y2_ref, sums_ref):
    h = jnp.maximum(y1_ref[0] * sc_ref[...] + sh_ref[...], 0.0)
    y2 = lax.dot_general(h, w2_ref[...], (((1,), (1,)), ((), ())),
                         preferred_element_type=jnp.float32) + b2_ref[...]
    y2_ref[0] = y2
    s1 = jnp.sum(y2, axis=0, keepdims=True)
    s2 = jnp.sum(y2 * y2, axis=0, keepdims=True)
    sums_ref[0, 0] = jnp.concatenate(
        [s1, s2, jnp.zeros((6, H2), jnp.float32)], axis=0)


def _k3_body(y2_ref, sc_ref, sh_ref, out_ref):
    out_ref[0] = jnp.maximum(y2_ref[0] * sc_ref[...] + sh_ref[...], 0.0)


@jax.jit
def kernel(xyz1, xyz2, points1, points2, W1, b1, g1, be1, W2, b2, g2, be2):
    nblk1 = N // NB1
    nblk2 = N // NB2

    dist = -2.0 * jnp.einsum('bnc,bmc->bnm', xyz1, xyz2)
    dist = dist + jnp.sum(xyz1 ** 2, axis=-1)[:, :, None]
    dist = dist + jnp.sum(xyz2 ** 2, axis=-1)[:, None, :]

    W1a = W1[:, :D1]
    W1b = W1[:, D1:]

    wsel, isel = pl.pallas_call(
        _k1sel_body,
        grid=(B, nblk1),
        in_specs=[pl.BlockSpec((1, NB1, S), lambda b, i: (b, i, 0))],
        out_specs=[
            pl.BlockSpec((1, 1, 8, NB1), lambda b, i: (b, i, 0, 0)),
            pl.BlockSpec((1, 1, 8, NB1), lambda b, i: (b, i, 0, 0)),
        ],
        out_shape=[
            jax.ShapeDtypeStruct((B, nblk1, 8, NB1), jnp.float32),
            jax.ShapeDtypeStruct((B, nblk1, 8, NB1), jnp.float32),
        ],
    )(dist)

    idx_all = jnp.concatenate(
        [isel[:, :, k, :].reshape(B * N) for k in range(3)]).astype(jnp.int32)
    table = jnp.pad(points2.reshape(B * S, D2), ((0, 0), (0, DPAD - D2)))
    rows = _sc_gather(table, idx_all)
    feats = rows[:, :D2].reshape(3, B, N, D2)
    wq = [wsel[:, :, k, :].reshape(B, N, 1) for k in range(3)]

    y1, sums1 = pl.pallas_call(
        _k1b_body,
        grid=(B, nblk1),
        in_specs=[
            pl.BlockSpec((1, NB1, D2), lambda b, i: (b, i, 0)),
            pl.BlockSpec((1, NB1, D2), lambda b, i: (b, i, 0)),
            pl.BlockSpec((1, NB1, D2), lambda b, i: (b, i, 0)),
            pl.BlockSpec((1, NB1, 1), lambda b, i: (b, i, 0)),
            pl.BlockSpec((1, NB1, 1), lambda b, i: (b, i, 0)),
            pl.BlockSpec((1, NB1, 1), lambda b, i: (b, i, 0)),
            pl.BlockSpec((1, NB1, D1), lambda b, i: (b, i, 0)),
            pl.BlockSpec((H1, D1), lambda b, i: (0, 0)),
            pl.BlockSpec((H1, D2), lambda b, i: (0, 0)),
            pl.BlockSpec((1, H1), lambda b, i: (0, 0)),
        ],
        out_specs=[
            pl.BlockSpec((1, NB1, H1), lambda b, i: (b, i, 0)),
            pl.BlockSpec((1, 1, 8, H1), lambda b, i: (b, i, 0, 0)),
        ],
        out_shape=[
            jax.ShapeDtypeStruct((B, N, H1), jnp.float32),
            jax.ShapeDtypeStruct((B, nblk1, 8, H1), jnp.float32),
        ],
    )(feats[0], feats[1], feats[2], wq[0], wq[1], wq[2], points1,
      W1a, W1b, b1.reshape(1, H1))

    cnt = float(B * N)
    t = jnp.sum(sums1, axis=(0, 1))
    mean1, ex2 = t[0] / cnt, t[1] / cnt
    var1 = ex2 - mean1 * mean1
    sc1 = g1 / jnp.sqrt(var1 + 1e-5)
    sh1 = be1 - mean1 * sc1

    y2, sums2 = pl.pallas_call(
        _k2_body,
        grid=(B, nblk2),
        in_specs=[
            pl.BlockSpec((1, NB2, H1), lambda b, i: (b, i, 0)),
            pl.BlockSpec((1, H1), lambda b, i: (0, 0)),
            pl.BlockSpec((1, H1), lambda b, i: (0, 0)),
            pl.BlockSpec((H2, H1), lambda b, i: (0, 0)),
            pl.BlockSpec((1, H2), lambda b, i: (0, 0)),
        ],
        out_specs=[
            pl.BlockSpec((1, NB2, H2), lambda b, i: (b, i, 0)),
            pl.BlockSpec((1, 1, 8, H2), lambda b, i: (b, i, 0, 0)),
        ],
        out_shape=[
            jax.ShapeDtypeStruct((B, N, H2), jnp.float32),
            jax.ShapeDtypeStruct((B, nblk2, 8, H2), jnp.float32),
        ],
    )(y1, sc1.reshape(1, H1), sh1.reshape(1, H1), W2, b2.reshape(1, H2))

    t = jnp.sum(sums2, axis=(0, 1))
    mean2, ex2 = t[0] / cnt, t[1] / cnt
    var2 = ex2 - mean2 * mean2
    sc2 = g2 / jnp.sqrt(var2 + 1e-5)
    sh2 = be2 - mean2 * sc2

    out = pl.pallas_call(
        _k3_body,
        grid=(B, nblk2),
        in_specs=[
            pl.BlockSpec((1, NB2, H2), lambda b, i: (b, i, 0)),
            pl.BlockSpec((1, H2), lambda b, i: (0, 0)),
            pl.BlockSpec((1, H2), lambda b, i: (0, 0)),
        ],
        out_specs=pl.BlockSpec((1, NB2, H2), lambda b, i: (b, i, 0)),
        out_shape=jax.ShapeDtypeStruct((B, N, H2), jnp.float32),
    )(y2, sc2.reshape(1, H2), sh2.reshape(1, H2))

    return out


# speedup vs baseline: 1.4000x; 1.0261x over previous
import functools

import jax
import jax.numpy as jnp
from jax import lax
from jax.experimental import pallas as pl
from jax.experimental.pallas import tpu as pltpu
from jax.experimental.pallas import tpu_sc as plsc

B, N, S = 4, 8192, 2048
D1, D2 = 16, 32
H1, H2 = 64, 64

NB1 = 512
NB2 = 2048
NH = N // 2
NSELH = 3 * B * NH
DPAD = 128
SC_CHUNK = 128


def _k1sel_body(dist_ref, wout_ref, iout_ref):
    d = dist_ref[0]
    iota = lax.broadcasted_iota(jnp.int32, (NB1, S), 1).astype(jnp.float32)
    base = (pl.program_id(0) * S).astype(jnp.int32)
    recips = []
    idxs = []
    for k in range(3):
        vmin = jnp.min(d, axis=1, keepdims=True)
        t = jnp.where(d == vmin, iota, float(S))
        imin = jnp.min(t, axis=1, keepdims=True)
        recips.append(1.0 / (vmin + 1e-8))
        idxs.append(imin + base.astype(jnp.float32))
        if k < 2:
            d = jnp.where(t == imin, jnp.inf, d)
    norm = (recips[0] + recips[1]) + recips[2]
    z = jnp.zeros((NB1, 5), jnp.float32)
    wout_ref[0, 0] = jnp.concatenate(
        [recips[0] / norm, recips[1] / norm, recips[2] / norm, z], axis=1).T
    iout_ref[0, 0] = jnp.concatenate(idxs + [z], axis=1).T


_SC_INFO = plsc.get_sparse_core_info()
_NW = _SC_INFO.num_cores * _SC_INFO.num_subcores


def _sc_gather_body(table_hbm, idx_hbm, out_hbm, idx_v, rows_v, sem):
    wid = lax.axis_index("s") * _SC_INFO.num_cores + lax.axis_index("c")
    per_w = NSELH // _NW
    n_chunks = per_w // SC_CHUNK
    for j in range(n_chunks):
        base = wid * per_w + j * SC_CHUNK
        pltpu.sync_copy(idx_hbm.at[pl.ds(base, SC_CHUNK)], idx_v)
        pltpu.async_copy(table_hbm.at[idx_v], rows_v, sem).wait()
        pltpu.sync_copy(rows_v, out_hbm.at[pl.ds(base, SC_CHUNK)])


_sc_gather = functools.partial(
    pl.kernel,
    mesh=plsc.VectorSubcoreMesh(core_axis_name="c", subcore_axis_name="s"),
    out_type=jax.ShapeDtypeStruct((NSELH, DPAD), jnp.float32),
    scratch_types=[
        pltpu.VMEM((SC_CHUNK,), jnp.int32),
        pltpu.VMEM((SC_CHUNK, DPAD), jnp.float32),
        pltpu.SemaphoreType.DMA,
    ],
)(_sc_gather_body)


def _k1b_body(f0_ref, f1_ref, f2_ref, w0_ref, w1_ref, w2_ref, p1_ref,
              w1a_ref, w1b_ref, b1_ref, y1_ref, sums_ref):
    f0 = f0_ref[0, 0][:, :D2]
    f1 = f1_ref[0, 0][:, :D2]
    f2 = f2_ref[0, 0][:, :D2]
    interp = ((w0_ref[0] * f0 + w1_ref[0] * f1)
              + w2_ref[0] * f2)
    y1 = (lax.dot_general(p1_ref[0], w1a_ref[...], (((1,), (1,)), ((), ())),
                          preferred_element_type=jnp.float32)
          + lax.dot_general(interp, w1b_ref[...], (((1,), (1,)), ((), ())),
                            preferred_element_type=jnp.float32)
          + b1_ref[...])
    y1_ref[0] = y1
    s1 = jnp.sum(y1, axis=0, keepdims=True)
    s2 = jnp.sum(y1 * y1, axis=0, keepdims=True)
    sums_ref[0, 0] = jnp.concatenate(
        [s1, s2, jnp.zeros((6, H1), jnp.float32)], axis=0)


def _k2_body(y1_ref, sc_ref, sh_ref, w2_ref, b2_ref, y2_ref, sums_ref):
    h = jnp.maximum(y1_ref[0] * sc_ref[...] + sh_ref[...], 0.0)
    y2 = lax.dot_general(h, w2_ref[...], (((1,), (1,)), ((), ())),
                         preferred_element_type=jnp.float32) + b2_ref[...]
    y2_ref[0] = y2
    s1 = jnp.sum(y2, axis=0, keepdims=True)
    s2 = jnp.sum(y2 * y2, axis=0, keepdims=True)
    sums_ref[0, 0] = jnp.concatenate(
        [s1, s2, jnp.zeros((6, H2), jnp.float32)], axis=0)


def _k3_body(y2_ref, sc_ref, sh_ref, out_ref):
    out_ref[0] = jnp.maximum(y2_ref[0] * sc_ref[...] + sh_ref[...], 0.0)


@jax.jit
def kernel(xyz1, xyz2, points1, points2, W1, b1, g1, be1, W2, b2, g2, be2):
    nblk2 = N // NB2

    dist = -2.0 * jnp.einsum('bnc,bmc->bnm', xyz1, xyz2)
    dist = dist + jnp.sum(xyz1 ** 2, axis=-1)[:, :, None]
    dist = dist + jnp.sum(xyz2 ** 2, axis=-1)[:, None, :]

    W1a = W1[:, :D1]
    W1b = W1[:, D1:]

    nblkh = NH // NB1
    table = jnp.pad(points2.reshape(B * S, D2), ((0, 0), (0, DPAD - D2)))

    sel = []
    for h in range(2):
        off = h * nblkh
        sel.append(pl.pallas_call(
            _k1sel_body,
            grid=(B, nblkh),
            in_specs=[pl.BlockSpec((1, NB1, S),
                                   lambda b, i, off=off: (b, i + off, 0))],
            out_specs=[
                pl.BlockSpec((1, 1, 8, NB1), lambda b, i: (b, i, 0, 0)),
                pl.BlockSpec((1, 1, 8, NB1), lambda b, i: (b, i, 0, 0)),
            ],
            out_shape=[
                jax.ShapeDtypeStruct((B, nblkh, 8, NB1), jnp.float32),
                jax.ShapeDtypeStruct((B, nblkh, 8, NB1), jnp.float32),
            ],
        )(dist))

    halves = []
    for h, (wsel, isel) in enumerate(sel):
        idx = jnp.concatenate(
            [isel[:, :, k, :].reshape(B * NH) for k in range(3)]
        ).astype(jnp.int32)
        rows = _sc_gather(table, idx)
        feats = rows.reshape(3, B, NH, DPAD)
        wq = [wsel[:, :, k, :].reshape(B, NH, 1) for k in range(3)]
        off = h * nblkh
        halves.append(pl.pallas_call(
            _k1b_body,
            grid=(B, nblkh),
            in_specs=[
                pl.BlockSpec((1, 1, NB1, DPAD), lambda b, i: (0, b, i, 0)),
                pl.BlockSpec((1, 1, NB1, DPAD), lambda b, i: (1, b, i, 0)),
                pl.BlockSpec((1, 1, NB1, DPAD), lambda b, i: (2, b, i, 0)),
                pl.BlockSpec((1, NB1, 1), lambda b, i: (b, i, 0)),
                pl.BlockSpec((1, NB1, 1), lambda b, i: (b, i, 0)),
                pl.BlockSpec((1, NB1, 1), lambda b, i: (b, i, 0)),
                pl.BlockSpec((1, NB1, D1),
                             lambda b, i, off=off: (b, i + off, 0)),
                pl.BlockSpec((H1, D1), lambda b, i: (0, 0)),
                pl.BlockSpec((H1, D2), lambda b, i: (0, 0)),
                pl.BlockSpec((1, H1), lambda b, i: (0, 0)),
            ],
            out_specs=[
                pl.BlockSpec((1, NB1, H1), lambda b, i: (b, i, 0)),
                pl.BlockSpec((1, 1, 8, H1), lambda b, i: (b, i, 0, 0)),
            ],
            out_shape=[
                jax.ShapeDtypeStruct((B, NH, H1), jnp.float32),
                jax.ShapeDtypeStruct((B, nblkh, 8, H1), jnp.float32),
            ],
        )(feats, feats, feats, wq[0], wq[1], wq[2], points1,
          W1a, W1b, b1.reshape(1, H1)))

    y1 = jnp.concatenate([halves[0][0], halves[1][0]], axis=1)

    cnt = float(B * N)
    t = jnp.sum(halves[0][1], axis=(0, 1)) + jnp.sum(halves[1][1], axis=(0, 1))
    mean1, ex2 = t[0] / cnt, t[1] / cnt
    var1 = ex2 - mean1 * mean1
    sc1 = g1 / jnp.sqrt(var1 + 1e-5)
    sh1 = be1 - mean1 * sc1

    y2, sums2 = pl.pallas_call(
        _k2_body,
        grid=(B, nblk2),
        in_specs=[
            pl.BlockSpec((1, NB2, H1), lambda b, i: (b, i, 0)),
            pl.BlockSpec((1, H1), lambda b, i: (0, 0)),
            pl.BlockSpec((1, H1), lambda b, i: (0, 0)),
            pl.BlockSpec((H2, H1), lambda b, i: (0, 0)),
            pl.BlockSpec((1, H2), lambda b, i: (0, 0)),
        ],
        out_specs=[
            pl.BlockSpec((1, NB2, H2), lambda b, i: (b, i, 0)),
            pl.BlockSpec((1, 1, 8, H2), lambda b, i: (b, i, 0, 0)),
        ],
        out_shape=[
            jax.ShapeDtypeStruct((B, N, H2), jnp.float32),
            jax.ShapeDtypeStruct((B, nblk2, 8, H2), jnp.float32),
        ],
    )(y1, sc1.reshape(1, H1), sh1.reshape(1, H1), W2, b2.reshape(1, H2))

    t = jnp.sum(sums2, axis=(0, 1))
    mean2, ex2 = t[0] / cnt, t[1] / cnt
    var2 = ex2 - mean2 * mean2
    sc2 = g2 / jnp.sqrt(var2 + 1e-5)
    sh2 = be2 - mean2 * sc2

    out = pl.pallas_call(
        _k3_body,
        grid=(B, nblk2),
        in_specs=[
            pl.BlockSpec((1, NB2, H2), lambda b, i: (b, i, 0)),
            pl.BlockSpec((1, H2), lambda b, i: (0, 0)),
            pl.BlockSpec((1, H2), lambda b, i: (0, 0)),
        ],
        out_specs=pl.BlockSpec((1, NB2, H2), lambda b, i: (b, i, 0)),
        out_shape=jax.ShapeDtypeStruct((B, N, H2), jnp.float32),
    )(y2, sc2.reshape(1, H2), sh2.reshape(1, H2))

    return out
